# Initial kernel scaffold; baseline (speedup 1.0000x reference)
import jax
import jax.numpy as jnp
from jax.experimental import pallas as pl


def kernel(x, edge_index, batch, W1, b1, W2, b2, W3, b3):
    raise NotImplementedError("write your pallas kernel here")


# speedup vs baseline: 29.1262x; 29.1262x over previous
import functools

import jax
import jax.numpy as jnp
from jax import lax
from jax.experimental import pallas as pl
from jax.experimental.pallas import tpu as pltpu
from jax.experimental.pallas import tpu_sc as plsc

N = 10000
E = 320000
D = 128
H = 32
B = 64

NC = 2
NS = 16
NW = NC * NS

N_PAD = 10240
ROWS_PER_TILE = N_PAD // NS
EPW = E // NW
EPW_PAD = 10240
CHUNK = 128
NCHUNK = EPW_PAD // CHUNK

_f32 = jnp.float32
_i32 = jnp.int32

def _mesh():
    return plsc.VectorSubcoreMesh(
        core_axis_name="c", subcore_axis_name="s",
        num_cores=NC, num_subcores=NS)


DW = 16


@functools.cache
def _deg_kernel_fn():
    return pl.kernel(
        _deg_body,
        out_type=jax.ShapeDtypeStruct((NC, N_PAD, DW), _f32),
        mesh=_mesh(),
        compiler_params=pltpu.CompilerParams(use_tc_tiling_on_sc=False),
        scratch_types=[
            pltpu.VMEM((NCHUNK, CHUNK), _i32),
            pltpu.VMEM((CHUNK, DW), _f32),
            pltpu.VMEM((CHUNK, DW), _f32),
            pltpu.VMEM_SHARED((N_PAD, DW), _f32),
        ],
    )


def _deg_body(dst_hbm, deg_out, idx_v, ones_v, zero_v, acc_sh):
    c = lax.axis_index("c")
    s = lax.axis_index("s")
    pltpu.sync_copy(dst_hbm.at[c, s], idx_v)
    for i in range(CHUNK):
        ones_v[i, pl.ds(0, DW)] = jnp.ones((DW,), _f32)
        zero_v[i, pl.ds(0, DW)] = jnp.zeros((DW,), _f32)
    r0 = s * ROWS_PER_TILE
    for k in range(ROWS_PER_TILE // CHUNK):
        pltpu.sync_copy(zero_v, acc_sh.at[pl.ds(r0 + k * CHUNK, CHUNK)])
    plsc.subcore_barrier()

    def body(j, carry):
        pltpu.sync_copy(ones_v, acc_sh.at[idx_v.at[j]], add=True)
        return carry

    lax.fori_loop(0, NCHUNK, body, 0)
    plsc.subcore_barrier()
    pltpu.sync_copy(acc_sh.at[pl.ds(r0, ROWS_PER_TILE)],
                    deg_out.at[c, pl.ds(r0, ROWS_PER_TILE)])


@functools.cache
def _edge_kernel_fn():
    return pl.kernel(
        _edge_body,
        out_type=jax.ShapeDtypeStruct((NC, N_PAD, H), _f32),
        mesh=_mesh(),
        compiler_params=pltpu.CompilerParams(use_tc_tiling_on_sc=False),
        scratch_types=[
            pltpu.VMEM((NCHUNK, CHUNK), _i32),
            pltpu.VMEM((NCHUNK, CHUNK), _i32),
            pltpu.VMEM((CHUNK, H), _f32),
            pltpu.VMEM((CHUNK, H), _f32),
            pltpu.VMEM_SHARED((N_PAD, H), _f32),
            pltpu.VMEM_SHARED((N_PAD, H), _f32),
        ],
    )


def _edge_body(p_hbm, src_hbm, dst_hbm, out_hbm,
               idxs_v, idxd_v, rows_v, zero_v, p_sh, acc_sh):
    c = lax.axis_index("c")
    s = lax.axis_index("s")
    pltpu.sync_copy(src_hbm.at[c, s], idxs_v)
    pltpu.sync_copy(dst_hbm.at[c, s], idxd_v)
    for i in range(CHUNK):
        zero_v[i, pl.ds(0, 16)] = jnp.zeros((16,), _f32)
        zero_v[i, pl.ds(16, 16)] = jnp.zeros((16,), _f32)
    r0 = s * ROWS_PER_TILE
    pltpu.sync_copy(p_hbm.at[pl.ds(r0, ROWS_PER_TILE)],
                    p_sh.at[pl.ds(r0, ROWS_PER_TILE)])
    for k in range(ROWS_PER_TILE // CHUNK):
        pltpu.sync_copy(zero_v, acc_sh.at[pl.ds(r0 + k * CHUNK, CHUNK)])
    plsc.subcore_barrier()

    def body(j, carry):
        pltpu.sync_copy(p_sh.at[idxs_v.at[j]], rows_v)
        pltpu.sync_copy(rows_v, acc_sh.at[idxd_v.at[j]], add=True)
        return carry

    lax.fori_loop(0, NCHUNK, body, 0)
    plsc.subcore_barrier()
    pltpu.sync_copy(acc_sh.at[pl.ds(r0, ROWS_PER_TILE)],
                    out_hbm.at[c, pl.ds(r0, ROWS_PER_TILE)])


def _tc1_body(x_ref, w1_ref, degs_ref, p1_ref, dinv_ref):
    degs = degs_ref[...]
    d = degs[0, :, 0:1] + degs[1, :, 0:1] + 1.0
    dinv = lax.rsqrt(jnp.maximum(d, 1.0))
    dinv_ref[...] = dinv
    h = jnp.dot(x_ref[...], w1_ref[...], preferred_element_type=_f32)
    p1_ref[...] = h * dinv


def _tc2_body(sp_ref, p1_ref, dinv_ref, w2_ref, b1_ref, p2_ref):
    sp = sp_ref[...]
    dinv = dinv_ref[...]
    h1 = jnp.maximum(dinv * (sp[0] + sp[1] + p1_ref[...]) + b1_ref[...], 0.0)
    p2_ref[...] = jnp.dot(h1, w2_ref[...], preferred_element_type=_f32) * dinv


def _tc3a_body(sp_ref, p2_ref, dinv_ref, b2_ref, batch_ref, h2_ref, mean_ref):
    sp = sp_ref[...]
    dinv = dinv_ref[...]
    h2 = jnp.maximum(dinv * (sp[0] + sp[1] + p2_ref[...]) + b2_ref[...], 0.0)
    h2_ref[...] = h2
    bt = batch_ref[...]
    seg = lax.broadcasted_iota(_i32, (1, B), 1)
    onehot = (bt == seg).astype(_f32)
    dn = (((0,), (0,)), ((), ()))
    ssum = lax.dot_general(onehot, h2, dn, preferred_element_type=_f32)
    cnt = lax.dot_general(onehot, jnp.ones((N_PAD, 1), _f32), dn,
                          preferred_element_type=_f32)
    mean_ref[...] = ssum / jnp.maximum(cnt, 1.0)


def _tc3b_body(h2_ref, batch_ref, mean_ref, w3_ref, b3_ref, out_ref):
    b = pl.program_id(0)
    neg = jnp.float32(-jnp.inf)
    mask = batch_ref[...] == b
    mx = jnp.max(jnp.where(mask, h2_ref[...], neg), axis=0, keepdims=True)
    g = jnp.concatenate([mean_ref[0], mx], axis=1)
    out_ref[0] = (
        jnp.dot(g, w3_ref[...], preferred_element_type=_f32) + b3_ref[...])


def _tc_call(body, out_shape, *args):
    return pl.pallas_call(body, out_shape=out_shape)(*args)


def _tc3b_call(h2, batch_pad, mean, W3, b3r):
    full = lambda s: pl.BlockSpec(s, lambda b: (0,) * len(s))
    return pl.pallas_call(
        _tc3b_body,
        grid=(B,),
        in_specs=[
            full((N_PAD, H)),
            full((N_PAD, 1)),
            pl.BlockSpec((1, 1, H), lambda b: (b, 0, 0)),
            full((2 * H, 2)),
            full((1, 2)),
        ],
        out_specs=pl.BlockSpec((1, 1, 2), lambda b: (b, 0, 0)),
        out_shape=jax.ShapeDtypeStruct((B, 1, 2), _f32),
    )(h2, batch_pad, mean.reshape(B, 1, H), W3, b3r).reshape(B, 2)


@jax.jit
def kernel(x, edge_index, batch, W1, b1, W2, b2, W3, b3):
    x_pad = jnp.concatenate(
        [x, jnp.zeros((N_PAD - N, D), _f32)], axis=0)

    pad_idx = N + (jnp.arange(EPW_PAD - EPW, dtype=_i32) % (N_PAD - N))
    pad_blk = jnp.broadcast_to(pad_idx, (NW, EPW_PAD - EPW))
    src = jnp.concatenate(
        [edge_index[0].reshape(NW, EPW), pad_blk], axis=1
    ).reshape(NC, NS, NCHUNK, CHUNK)
    dst = jnp.concatenate(
        [edge_index[1].reshape(NW, EPW), pad_blk], axis=1
    ).reshape(NC, NS, NCHUNK, CHUNK)

    batch_pad = jnp.concatenate(
        [batch, jnp.full((N_PAD - N,), B, _i32)]).reshape(N_PAD, 1)

    degs = _deg_kernel_fn()(dst)

    p1, dinv = _tc_call(
        _tc1_body,
        (jax.ShapeDtypeStruct((N_PAD, H), _f32),
         jax.ShapeDtypeStruct((N_PAD, 1), _f32)),
        x_pad, W1, degs)

    s1 = _edge_kernel_fn()(p1, src, dst)

    p2 = _tc_call(
        _tc2_body, jax.ShapeDtypeStruct((N_PAD, H), _f32),
        s1, p1, dinv, W2, b1.reshape(1, H))

    s2 = _edge_kernel_fn()(p2, src, dst)

    h2, mean = _tc_call(
        _tc3a_body,
        (jax.ShapeDtypeStruct((N_PAD, H), _f32),
         jax.ShapeDtypeStruct((B, H), _f32)),
        s2, p2, dinv, b2.reshape(1, H), batch_pad)

    return _tc3b_call(h2, batch_pad, mean, W3, b3.reshape(1, 2))
